# baseline (speedup 1.0000x reference)
import functools

import jax
import jax.numpy as jnp
from jax import lax
from jax.experimental import pallas as pl
from jax.experimental.pallas import tpu as pltpu
from jax.experimental.pallas import tpu_sc as plsc

_B = 16384
_D = 64
_NW = 32
_BPW = _B // _NW
_CH = 128
_RCH = 256
_NCH = _BPW // _RCH


def _build_sc_gather():
    mesh = plsc.VectorSubcoreMesh(core_axis_name="c", subcore_axis_name="s")

    @functools.partial(
        pl.kernel,
        out_type=(
            jax.ShapeDtypeStruct((_B, 2 * _D), jnp.float32),
            jax.ShapeDtypeStruct((_B, 2 * _D), jnp.float32),
        ),
        mesh=mesh,
        scratch_types=[
            pltpu.VMEM((_BPW,), jnp.int32),
            pltpu.VMEM((_BPW,), jnp.int32),
            pltpu.VMEM((_RCH, 2 * _D), jnp.float32),
            pltpu.VMEM((_RCH, 2 * _D), jnp.float32),
            pltpu.SemaphoreType.DMA((4,)),
        ],
    )
    def gather(uidx_hbm, iidx_hbm, utab_hbm, itab_hbm, gu_hbm, gi_hbm,
               uidx_v, iidx_v, urows_v, irows_v, sems):
        wid = lax.axis_index("s") * 2 + lax.axis_index("c")
        base = wid * _BPW
        pltpu.sync_copy(uidx_hbm.at[pl.ds(base, _BPW)], uidx_v)
        pltpu.sync_copy(iidx_hbm.at[pl.ds(base, _BPW)], iidx_v)

        for c in range(_NCH):
            def streams(c=c):
                out = []
                for s in range(_RCH // _CH):
                    off = c * _RCH + s * _CH
                    out.append(pltpu.make_async_copy(
                        utab_hbm.at[uidx_v.at[pl.ds(off, _CH)]],
                        urows_v.at[pl.ds(s * _CH, _CH)],
                        sems.at[2 * s]))
                    out.append(pltpu.make_async_copy(
                        itab_hbm.at[iidx_v.at[pl.ds(off, _CH)]],
                        irows_v.at[pl.ds(s * _CH, _CH)],
                        sems.at[2 * s + 1]))
                return out

            for h in streams():
                h.start()
            for h in streams():
                h.wait()
            pltpu.sync_copy(urows_v, gu_hbm.at[pl.ds(base + c * _RCH, _RCH)])
            pltpu.sync_copy(irows_v, gi_hbm.at[pl.ds(base + c * _RCH, _RCH)])

    return gather


_sc_gather = _build_sc_gather()

_MM_BLK = 1024


def _mm_body(gu_ref, gi_ref, uh_ref, ih_ref, w1_ref, w2_ref, b_ref, o_ref):
    uh = uh_ref[...] == 1
    ih = ih_ref[...] == 1
    ux = jnp.where(uh, gu_ref[:, _D:], gu_ref[:, :_D])
    ix = jnp.where(ih, gi_ref[:, _D:], gi_ref[:, :_D])
    acc = jnp.dot(ux, w1_ref[...], preferred_element_type=jnp.float32)
    acc = acc + jnp.dot(ix, w2_ref[...], preferred_element_type=jnp.float32)
    o_ref[...] = acc + b_ref[...]


def _tc_matmul(gu, gi, uh, ih, w1t, w2t, b2):
    return pl.pallas_call(
        _mm_body,
        grid=(_B // _MM_BLK,),
        in_specs=[
            pl.BlockSpec((_MM_BLK, 2 * _D), lambda i: (i, 0)),
            pl.BlockSpec((_MM_BLK, 2 * _D), lambda i: (i, 0)),
            pl.BlockSpec((_MM_BLK, 1), lambda i: (i, 0)),
            pl.BlockSpec((_MM_BLK, 1), lambda i: (i, 0)),
            pl.BlockSpec((_D, _D), lambda i: (0, 0)),
            pl.BlockSpec((_D, _D), lambda i: (0, 0)),
            pl.BlockSpec((1, _D), lambda i: (0, 0)),
        ],
        out_specs=pl.BlockSpec((_MM_BLK, _D), lambda i: (i, 0)),
        out_shape=jax.ShapeDtypeStruct((_B, _D), jnp.float32),
    )(gu, gi, uh, ih, w1t, w2t, b2)


def kernel(x, user_table, item_table, W, b):
    uidx = x[:, 0]
    iidx = x[:, 1]
    utab2 = user_table.reshape(-1, 2 * _D)
    itab2 = item_table.reshape(-1, 2 * _D)
    gu, gi = _sc_gather(uidx >> 1, iidx >> 1, utab2, itab2)
    w1t = W[:, :_D].T
    w2t = W[:, _D:].T
    return _tc_matmul(gu, gi, (uidx & 1).reshape(_B, 1),
                      (iidx & 1).reshape(_B, 1), w1t, w2t, b.reshape(1, _D))


# speedup vs baseline: 1.0052x; 1.0052x over previous
import functools

import jax
import jax.numpy as jnp
from jax import lax
from jax.experimental import pallas as pl
from jax.experimental.pallas import tpu as pltpu
from jax.experimental.pallas import tpu_sc as plsc

_B = 16384
_D = 64
_NW = 32
_BPW = _B // _NW
_CH = 128
_NCH = _BPW // _CH


def _build_sc_gather():
    mesh = plsc.VectorSubcoreMesh(core_axis_name="c", subcore_axis_name="s")

    @functools.partial(
        pl.kernel,
        out_type=(
            jax.ShapeDtypeStruct((_B, _D), jnp.float32),
            jax.ShapeDtypeStruct((_B, _D), jnp.float32),
        ),
        mesh=mesh,
        scratch_types=[
            pltpu.VMEM((_BPW,), jnp.int32),
            pltpu.VMEM((_BPW,), jnp.int32),
            pltpu.VMEM((_BPW, _D), jnp.float32),
            pltpu.VMEM((_BPW, _D), jnp.float32),
            pltpu.SemaphoreType.DMA((2 * _NCH,)),
        ],
        compiler_params=pltpu.CompilerParams(use_tc_tiling_on_sc=False),
    )
    def gather(uidx_hbm, iidx_hbm, utab_hbm, itab_hbm, ux_hbm, ix_hbm,
               uidx_v, iidx_v, urows_v, irows_v, sems):
        wid = lax.axis_index("s") * 2 + lax.axis_index("c")
        base = wid * _BPW
        pltpu.sync_copy(uidx_hbm.at[pl.ds(base, _BPW)], uidx_v)
        pltpu.sync_copy(iidx_hbm.at[pl.ds(base, _BPW)], iidx_v)

        def streams():
            out = []
            for s in range(_NCH):
                off = s * _CH
                out.append(pltpu.make_async_copy(
                    utab_hbm.at[uidx_v.at[pl.ds(off, _CH)]],
                    urows_v.at[pl.ds(off, _CH)],
                    sems.at[2 * s]))
                out.append(pltpu.make_async_copy(
                    itab_hbm.at[iidx_v.at[pl.ds(off, _CH)]],
                    irows_v.at[pl.ds(off, _CH)],
                    sems.at[2 * s + 1]))
            return out

        for h in streams():
            h.start()
        for h in streams():
            h.wait()

        pltpu.sync_copy(urows_v, ux_hbm.at[pl.ds(base, _BPW)])
        pltpu.sync_copy(irows_v, ix_hbm.at[pl.ds(base, _BPW)])

    return gather


_sc_gather = _build_sc_gather()

_MM_BLK = 1024


def _mm_body(ux_ref, ix_ref, w1_ref, w2_ref, b_ref, o_ref):
    acc = jnp.dot(ux_ref[...], w1_ref[...], preferred_element_type=jnp.float32)
    acc = acc + jnp.dot(ix_ref[...], w2_ref[...], preferred_element_type=jnp.float32)
    o_ref[...] = acc + b_ref[...]


def _tc_matmul(ux, ix, w1t, w2t, b2):
    return pl.pallas_call(
        _mm_body,
        grid=(_B // _MM_BLK,),
        in_specs=[
            pl.BlockSpec((_MM_BLK, _D), lambda i: (i, 0)),
            pl.BlockSpec((_MM_BLK, _D), lambda i: (i, 0)),
            pl.BlockSpec((_D, _D), lambda i: (0, 0)),
            pl.BlockSpec((_D, _D), lambda i: (0, 0)),
            pl.BlockSpec((1, _D), lambda i: (0, 0)),
        ],
        out_specs=pl.BlockSpec((_MM_BLK, _D), lambda i: (i, 0)),
        out_shape=jax.ShapeDtypeStruct((_B, _D), jnp.float32),
    )(ux, ix, w1t, w2t, b2)


def kernel(x, user_table, item_table, W, b):
    ux, ix = _sc_gather(x[:, 0], x[:, 1], user_table, item_table)
    w1t = W[:, :_D].T
    w2t = W[:, _D:].T
    return _tc_matmul(ux, ix, w1t, w2t, b.reshape(1, _D))


# speedup vs baseline: 1.4518x; 1.4444x over previous
import functools

import jax
import jax.numpy as jnp
from jax import lax
from jax.experimental import pallas as pl
from jax.experimental.pallas import tpu as pltpu
from jax.experimental.pallas import tpu_sc as plsc

_B = 16384
_D = 64
_NW = 32
_BPW = _B // _NW
_RCH = 128
_NCH = _BPW // _RCH
_RING = 8
_L = 16


def _build_sc_gather():
    mesh = plsc.VectorSubcoreMesh(core_axis_name="c", subcore_axis_name="s")

    @functools.partial(
        pl.kernel,
        out_type=(
            jax.ShapeDtypeStruct((_B, _D), jnp.float32),
            jax.ShapeDtypeStruct((_B, _D), jnp.float32),
        ),
        mesh=mesh,
        scratch_types=[
            pltpu.VMEM((_BPW,), jnp.int32),
            pltpu.VMEM((_BPW,), jnp.int32),
            pltpu.VMEM((_RCH, _D), jnp.float32),
            pltpu.VMEM((_RCH, _D), jnp.float32),
            pltpu.SemaphoreType.DMA((_RING,)),
            pltpu.SemaphoreType.DMA((_RING,)),
        ],
        compiler_params=pltpu.CompilerParams(needs_layout_passes=False),
    )
    def gather(uidx_hbm, iidx_hbm, utab_hbm, itab_hbm, ux_hbm, ix_hbm,
               uidx_v, iidx_v, urow_v, irow_v, usem, isem):
        wid = lax.axis_index("s") * 2 + lax.axis_index("c")
        base = wid * _BPW
        pltpu.sync_copy(uidx_hbm.at[pl.ds(base, _BPW)], uidx_v)
        pltpu.sync_copy(iidx_hbm.at[pl.ds(base, _BPW)], iidx_v)

        iota = jnp.arange(_L, dtype=jnp.int32)
        zero = jnp.zeros((_L,), dtype=jnp.int32)

        def fire(tab_hbm, row_v, sem, idx_s, r):
            pltpu.make_async_copy(
                tab_hbm.at[pl.ds(idx_s, 1)],
                row_v.at[pl.ds(r, 1)],
                sem.at[r % _RING]).start()

        def drain(tab_hbm, row_v, sem, r):
            pltpu.make_async_copy(
                tab_hbm.at[pl.ds(0, 1)],
                row_v.at[pl.ds(r, 1)],
                sem.at[r % _RING]).wait()

        def chunk_body(ch, _):
            off = ch * _RCH
            for g in range(_RCH // _L):
                u16 = uidx_v[pl.ds(off + g * _L, _L)]
                i16 = iidx_v[pl.ds(off + g * _L, _L)]
                for j in range(_L):
                    r = g * _L + j
                    us = jnp.max(jnp.where(iota == j, u16, zero))
                    is_ = jnp.max(jnp.where(iota == j, i16, zero))
                    if r >= _RING:
                        drain(utab_hbm, urow_v, usem, r - _RING)
                        drain(itab_hbm, irow_v, isem, r - _RING)
                    fire(utab_hbm, urow_v, usem, us, r)
                    fire(itab_hbm, irow_v, isem, is_, r)
            for r in range(_RCH - _RING, _RCH):
                drain(utab_hbm, urow_v, usem, r)
                drain(itab_hbm, irow_v, isem, r)
            pltpu.sync_copy(urow_v, ux_hbm.at[pl.ds(base + off, _RCH)])
            pltpu.sync_copy(irow_v, ix_hbm.at[pl.ds(base + off, _RCH)])
            return 0

        lax.fori_loop(0, _NCH, chunk_body, 0)

    return gather


_sc_gather = _build_sc_gather()

_MM_BLK = 1024


def _mm_body(ux_ref, ix_ref, w1_ref, w2_ref, b_ref, o_ref):
    acc = jnp.dot(ux_ref[...], w1_ref[...], preferred_element_type=jnp.float32)
    acc = acc + jnp.dot(ix_ref[...], w2_ref[...], preferred_element_type=jnp.float32)
    o_ref[...] = acc + b_ref[...]


def _tc_matmul(ux, ix, w1t, w2t, b2):
    return pl.pallas_call(
        _mm_body,
        grid=(_B // _MM_BLK,),
        in_specs=[
            pl.BlockSpec((_MM_BLK, _D), lambda i: (i, 0)),
            pl.BlockSpec((_MM_BLK, _D), lambda i: (i, 0)),
            pl.BlockSpec((_D, _D), lambda i: (0, 0)),
            pl.BlockSpec((_D, _D), lambda i: (0, 0)),
            pl.BlockSpec((1, _D), lambda i: (0, 0)),
        ],
        out_specs=pl.BlockSpec((_MM_BLK, _D), lambda i: (i, 0)),
        out_shape=jax.ShapeDtypeStruct((_B, _D), jnp.float32),
    )(ux, ix, w1t, w2t, b2)


def kernel(x, user_table, item_table, W, b):
    ux, ix = _sc_gather(x[:, 0], x[:, 1], user_table, item_table)
    w1t = W[:, :_D].T
    w2t = W[:, _D:].T
    return _tc_matmul(ux, ix, w1t, w2t, b.reshape(1, _D))
